# initial kernel scaffold (unmeasured)
import jax
import jax.numpy as jnp
from jax import lax
from jax.experimental import pallas as pl
from jax.experimental.pallas import tpu as pltpu


def kernel(
    x,
):
    def body(*refs):
        pass

    out_shape = jax.ShapeDtypeStruct(..., jnp.float32)
    return pl.pallas_call(body, out_shape=out_shape)(...)


# baseline (device time: 51151 ns/iter reference)
import jax
import jax.numpy as jnp
from jax import lax
from jax.experimental import pallas as pl
from jax.experimental.pallas import tpu as pltpu

N_DEV = 4


def kernel(x):
    _, m, n = x.shape
    c = m // N_DEV

    def body(x_ref, out_ref, sbuf, rbuf, agbuf, send_sems, recv_sems):
        p = lax.axis_index("i")
        left = lax.rem(p + (N_DEV - 1), N_DEV)
        right = lax.rem(p + 1, N_DEV)

        barrier_sem = pltpu.get_barrier_semaphore()
        for nbr in (left, right):
            pl.semaphore_signal(
                barrier_sem, inc=1,
                device_id=(nbr,), device_id_type=pl.DeviceIdType.MESH,
            )
        pl.semaphore_wait(barrier_sem, 2)

        def local_chunk(idx):
            return x_ref[0, pl.ds(idx * c, c), :].astype(jnp.bfloat16)

        sbuf[0] = local_chunk(p)
        for s in range(N_DEV - 1):
            rdma = pltpu.make_async_remote_copy(
                src_ref=sbuf.at[s],
                dst_ref=rbuf.at[s],
                send_sem=send_sems.at[s],
                recv_sem=recv_sems.at[s],
                device_id=(right,),
                device_id_type=pl.DeviceIdType.MESH,
            )
            rdma.start()
            rdma.wait()
            recv_idx = lax.rem(p + (N_DEV - 1 - s), N_DEV)
            partial = rbuf[s] + local_chunk(recv_idx)
            if s < N_DEV - 2:
                sbuf[s + 1] = partial
            else:
                agbuf[0] = partial

        own_idx = right
        out_ref[pl.ds(own_idx * c, c), :] = agbuf[0].astype(jnp.float32)
        for s in range(N_DEV - 1):
            rdma = pltpu.make_async_remote_copy(
                src_ref=agbuf.at[s],
                dst_ref=agbuf.at[s + 1],
                send_sem=send_sems.at[N_DEV - 1 + s],
                recv_sem=recv_sems.at[N_DEV - 1 + s],
                device_id=(right,),
                device_id_type=pl.DeviceIdType.MESH,
            )
            rdma.start()
            rdma.wait()
            got_idx = lax.rem(p + (N_DEV - s), N_DEV)
            out_ref[pl.ds(got_idx * c, c), :] = agbuf[s + 1].astype(jnp.float32)

    n_hops = 2 * (N_DEV - 1)
    return pl.pallas_call(
        body,
        out_shape=jax.ShapeDtypeStruct((m, n), jnp.float32),
        in_specs=[pl.BlockSpec(memory_space=pltpu.VMEM)],
        out_specs=pl.BlockSpec(memory_space=pltpu.VMEM),
        scratch_shapes=[
            pltpu.VMEM((N_DEV - 1, c, n), jnp.bfloat16),
            pltpu.VMEM((N_DEV - 1, c, n), jnp.bfloat16),
            pltpu.VMEM((N_DEV, c, n), jnp.bfloat16),
            pltpu.SemaphoreType.DMA((n_hops,)),
            pltpu.SemaphoreType.DMA((n_hops,)),
        ],
        compiler_params=pltpu.CompilerParams(collective_id=0),
    )(x)


# device time: 30627 ns/iter; 1.6701x vs baseline; 1.6701x over previous
import jax
import jax.numpy as jnp
from jax import lax
from jax.experimental import pallas as pl
from jax.experimental.pallas import tpu as pltpu

N_DEV = 4


def kernel(x):
    _, m, n = x.shape
    half = m // 2
    qtr = m // 4
    blk = m // 8

    def body(x_ref, out_ref, xb, recv_a1, recv_b1, recv_a2, recv_b2,
             acc_a, acc_b, g_a, g_b, ssem, rsem):
        p = lax.axis_index("i")
        q = p ^ 1
        r = 3 - p

        barrier_sem = pltpu.get_barrier_semaphore()
        for nbr in (q, r):
            pl.semaphore_signal(
                barrier_sem, inc=1,
                device_id=(nbr,), device_id_type=pl.DeviceIdType.MESH,
            )
        pl.semaphore_wait(barrier_sem, 2)

        j = jnp.where((p == 1) | (p == 2), 1, 0)
        k = p // 2
        jb = p // 2
        kb = p % 2
        fa_me = 2 * j + k
        fa_r = 2 * j + (1 - k)

        def rdma(src, dst, sem_idx, dev):
            return pltpu.make_async_remote_copy(
                src_ref=src, dst_ref=dst,
                send_sem=ssem.at[sem_idx], recv_sem=rsem.at[sem_idx],
                device_id=(dev,), device_id_type=pl.DeviceIdType.MESH,
            )

        xb[...] = x_ref[0].astype(jnp.bfloat16)

        a1 = rdma(xb.at[pl.ds((1 - j) * qtr, qtr)], recv_a1, 0, q)
        b1 = rdma(xb.at[pl.ds(half + (1 - jb) * qtr, qtr)], recv_b1, 1, r)
        a1.start()
        b1.start()

        a1.wait()
        acc_a[...] = xb[pl.ds(j * qtr, qtr), :] + recv_a1[...]
        a2 = rdma(acc_a.at[pl.ds((1 - k) * blk, blk)], recv_a2, 2, r)
        a2.start()

        b1.wait()
        acc_b[...] = xb[pl.ds(half + jb * qtr, qtr), :] + recv_b1[...]
        b2 = rdma(acc_b.at[pl.ds((1 - kb) * blk, blk)], recv_b2, 3, q)
        b2.start()

        a2.wait()
        red_a = acc_a[pl.ds(k * blk, blk), :] + recv_a2[...]
        g_a[pl.ds(fa_me * blk, blk), :] = red_a
        out_ref[pl.ds(fa_me * blk, blk), :] = red_a.astype(jnp.float32)
        a3 = rdma(g_a.at[pl.ds(fa_me * blk, blk)],
                  g_a.at[pl.ds(fa_me * blk, blk)], 4, r)
        a3.start()

        b2.wait()
        red_b = acc_b[pl.ds(kb * blk, blk), :] + recv_b2[...]
        g_b[pl.ds(p * blk, blk), :] = red_b
        out_ref[pl.ds(half + p * blk, blk), :] = red_b.astype(jnp.float32)
        b3 = rdma(g_b.at[pl.ds(p * blk, blk)],
                  g_b.at[pl.ds(p * blk, blk)], 5, q)
        b3.start()

        a3.wait()
        out_ref[pl.ds(fa_r * blk, blk), :] = (
            g_a[pl.ds(fa_r * blk, blk), :].astype(jnp.float32))
        a4 = rdma(g_a.at[pl.ds(j * qtr, qtr)],
                  g_a.at[pl.ds(j * qtr, qtr)], 6, q)
        a4.start()

        b3.wait()
        out_ref[pl.ds(half + q * blk, blk), :] = (
            g_b[pl.ds(q * blk, blk), :].astype(jnp.float32))
        b4 = rdma(g_b.at[pl.ds(jb * qtr, qtr)],
                  g_b.at[pl.ds(jb * qtr, qtr)], 7, r)
        b4.start()

        a4.wait()
        out_ref[pl.ds((1 - j) * qtr, qtr), :] = (
            g_a[pl.ds((1 - j) * qtr, qtr), :].astype(jnp.float32))

        b4.wait()
        out_ref[pl.ds(half + (1 - jb) * qtr, qtr), :] = (
            g_b[pl.ds((1 - jb) * qtr, qtr), :].astype(jnp.float32))

    return pl.pallas_call(
        body,
        out_shape=jax.ShapeDtypeStruct((m, n), jnp.float32),
        in_specs=[pl.BlockSpec(memory_space=pltpu.VMEM)],
        out_specs=pl.BlockSpec(memory_space=pltpu.VMEM),
        scratch_shapes=[
            pltpu.VMEM((m, n), jnp.bfloat16),
            pltpu.VMEM((qtr, n), jnp.bfloat16),
            pltpu.VMEM((qtr, n), jnp.bfloat16),
            pltpu.VMEM((blk, n), jnp.bfloat16),
            pltpu.VMEM((blk, n), jnp.bfloat16),
            pltpu.VMEM((qtr, n), jnp.bfloat16),
            pltpu.VMEM((qtr, n), jnp.bfloat16),
            pltpu.VMEM((half, n), jnp.bfloat16),
            pltpu.VMEM((half, n), jnp.bfloat16),
            pltpu.SemaphoreType.DMA((8,)),
            pltpu.SemaphoreType.DMA((8,)),
        ],
        compiler_params=pltpu.CompilerParams(collective_id=0),
    )(x)
